# baseline (speedup 1.0000x reference)
import functools

import jax
import jax.numpy as jnp
from jax import lax
from jax.experimental import pallas as pl
from jax.experimental.pallas import tpu as pltpu
from jax.experimental.pallas import tpu_sc as plsc

B, C, H, W = 2, 96, 256, 256
HW = H * W
G = 7
NTILES = 16
RPT = HW // NTILES
CHUNK = 128
SCHUNK = 512
NSUB = SCHUNK // CHUNK
HB2 = 8
HB3 = 32
UNROLL = 16
PR = HW // 128


def _prep_body(flow_ref, metric_ref, cw_ref, idx_ref):
    ii = lax.broadcasted_iota(jnp.int32, (PR, 128), 0)
    jj = lax.broadcasted_iota(jnp.int32, (PR, 128), 1)
    gy = (ii // 2).astype(jnp.float32)
    gx = ((ii % 2) * 128 + jj).astype(jnp.float32)
    fx = gx + flow_ref[0, 0, 0]
    fy = gy + flow_ref[0, 0, 1]
    x0 = jnp.floor(fx)
    y0 = jnp.floor(fy)
    x0i = x0.astype(jnp.int32)
    y0i = y0.astype(jnp.int32)
    wexp = jnp.exp(metric_ref[0, 0])
    for k in range(4):
        dx, dy = k % 2, k // 2
        xi = x0i + dx
        yi = y0i + dy
        wx = (x0 + 1.0 - fx) if dx == 0 else (fx - x0)
        wy = (y0 + 1.0 - fy) if dy == 0 else (fy - y0)
        valid = (xi >= 0) & (xi < W) & (yi >= 0) & (yi < H)
        cw_ref[0, 0, k] = jnp.where(valid, wx * wy * wexp, 0.0)
        idx_ref[0, 0, k] = jnp.clip(yi, 0, H - 1) * W + jnp.clip(xi, 0, W - 1)


def _prep(flow, metric2):
    return pl.pallas_call(
        _prep_body,
        grid=(B, 2),
        in_specs=[
            pl.BlockSpec((1, 1, 2, PR, 128), lambda b, d: (b, d, 0, 0, 0)),
            pl.BlockSpec((1, 1, PR, 128), lambda b, d: (b, d, 0, 0)),
        ],
        out_specs=[
            pl.BlockSpec((1, 1, 4, PR, 128), lambda b, d: (b, d, 0, 0, 0)),
            pl.BlockSpec((1, 1, 4, PR, 128), lambda b, d: (b, d, 0, 0, 0)),
        ],
        out_shape=[
            jax.ShapeDtypeStruct((B, 2, 4, PR, 128), jnp.float32),
            jax.ShapeDtypeStruct((B, 2, 4, PR, 128), jnp.int32),
        ],
    )(flow, metric2)


def _pm_body(f_ref, b_ref, out_ref):
    pad = jnp.where(
        lax.broadcasted_iota(jnp.int32, (32, HB2, W), 0) == 0, 1.0, 0.0)
    for d, src in enumerate((f_ref, b_ref)):
        full = jnp.concatenate([src[0], pad], axis=0)
        for h in range(HB2):
            out_ref[0, d, pl.ds(h * W, W), :] = full[:, h, :].T


def _pixel_major(ftensor, btensor):
    return pl.pallas_call(
        _pm_body,
        grid=(B, H // HB2),
        in_specs=[
            pl.BlockSpec((1, C, HB2, W), lambda b, hb: (b, 0, hb, 0)),
            pl.BlockSpec((1, C, HB2, W), lambda b, hb: (b, 0, hb, 0)),
        ],
        out_specs=pl.BlockSpec((1, 2, HB2 * W, 128),
                               lambda b, hb: (b, 0, hb, 0)),
        out_shape=jax.ShapeDtypeStruct((B, 2, HW, 128), jnp.float32),
    )(ftensor, btensor)


def _sc_splat_body(tpm_hbm, cw_hbm, idx_hbm, out_hbm,
                   rows_v, sb0, sb1, w_v, idx_v, zero_v,
                   acc_sh, sem0, sem1):
    b = lax.axis_index("c")
    s = lax.axis_index("s")
    base = s * RPT
    sbufs = (sb0, sb1)
    sems = (sem0, sem1)

    def zfill(i, _):
        zero_v[i, :] = jnp.zeros((16,), jnp.float32)
        return 0
    lax.fori_loop(0, SCHUNK, zfill, 0)

    def per_g(g, _):
        def zloop(z, _):
            pltpu.sync_copy(zero_v, acc_sh.at[pl.ds(base + z * SCHUNK, SCHUNK)])
            return 0
        lax.fori_loop(0, RPT // SCHUNK, zloop, 0)
        plsc.subcore_barrier()

        def per_dc(dc, _):
            d = dc // (RPT // SCHUNK)
            c2 = dc % (RPT // SCHUNK)
            off = base + c2 * SCHUNK
            pltpu.sync_copy(
                tpm_hbm.at[b, d, pl.ds(off, SCHUNK), pl.ds(g * 16, 16)],
                rows_v)
            pltpu.sync_copy(
                cw_hbm.at[b, d, :, pl.ds(off // 128, NSUB)], w_v)
            pltpu.sync_copy(
                idx_hbm.at[b, d, :, pl.ds(off // 128, NSUB)], idx_v)

            pend = [None, None]
            for k in range(4):
                sb = sbufs[k % 2]
                if pend[k % 2] is not None:
                    for dsc in pend[k % 2]:
                        dsc.wait()

                def scale16(i, _, sb=sb, k=k):
                    r = i * UNROLL
                    wv = w_v[k, r // 128, pl.ds(r % 128, UNROLL)]
                    for u in range(UNROLL):
                        sb[r + u, :] = rows_v[r + u, :] * wv[u]
                    return 0
                lax.fori_loop(0, SCHUNK // UNROLL, scale16, 0)

                pend[k % 2] = [
                    pltpu.async_copy(sb.at[pl.ds(j * CHUNK, CHUNK)],
                                     acc_sh.at[idx_v.at[k, j]],
                                     sems[k % 2], add=True)
                    for j in range(NSUB)
                ]

            for q in range(2):
                for dsc in pend[q]:
                    dsc.wait()
            return 0

        lax.fori_loop(0, 2 * (RPT // SCHUNK), per_dc, 0)
        plsc.subcore_barrier()

        def dloop(z, _):
            pltpu.sync_copy(acc_sh.at[pl.ds(base + z * SCHUNK, SCHUNK)],
                            out_hbm.at[b, g, pl.ds(base + z * SCHUNK, SCHUNK)])
            return 0
        lax.fori_loop(0, RPT // SCHUNK, dloop, 0)
        plsc.subcore_barrier()
        return 0

    lax.fori_loop(0, G, per_g, 0)


@functools.cache
def _sc_splat_call():
    mesh = plsc.VectorSubcoreMesh(core_axis_name="c", subcore_axis_name="s")
    return pl.kernel(
        _sc_splat_body,
        out_type=jax.ShapeDtypeStruct((B, G, HW, 16), jnp.float32),
        mesh=mesh,
        scratch_types=[
            pltpu.VMEM((SCHUNK, 16), jnp.float32),
            pltpu.VMEM((SCHUNK, 16), jnp.float32),
            pltpu.VMEM((SCHUNK, 16), jnp.float32),
            pltpu.VMEM((4, NSUB, 128), jnp.float32),
            pltpu.VMEM((4, NSUB, 128), jnp.int32),
            pltpu.VMEM((SCHUNK, 16), jnp.float32),
            pltpu.VMEM_SHARED((HW, 16), jnp.float32),
            pltpu.SemaphoreType.DMA,
            pltpu.SemaphoreType.DMA,
        ],
        compiler_params=pltpu.CompilerParams(use_tc_tiling_on_sc=False),
    )


PB = 512


def _norm_body(acc_ref, nacc_ref, out_ref):
    lane = lax.broadcasted_iota(jnp.int32, (PB, 128), 1) % 16
    nb = nacc_ref[0, 0]
    for sh in (1, 2, 4, 8):
        nb = jnp.where(lane >= sh, jnp.roll(nb, sh, axis=1), nb)
    nb = jnp.where(nb == 0.0, 1.0, nb)
    out_ref[0, 0] = acc_ref[0, 0] / nb


def _norm(acc8):
    return pl.pallas_call(
        _norm_body,
        grid=(B, 6, (HW * 16 // 128) // PB),
        in_specs=[
            pl.BlockSpec((1, 1, PB, 128), lambda b, g, p: (b, g, p, 0)),
            pl.BlockSpec((1, 1, PB, 128), lambda b, g, p: (b, 6, p, 0)),
        ],
        out_specs=pl.BlockSpec((1, 1, PB, 128), lambda b, g, p: (b, g, p, 0)),
        out_shape=jax.ShapeDtypeStruct((B, 6, HW * 16 // 128, 128),
                                       jnp.float32),
    )(acc8, acc8)


def kernel(ftensor, fflow, fmetric, btensor, bflow, bmetric, t, N):
    alpha = (t / N).astype(jnp.float32)
    af = jnp.stack([1.0 - alpha, alpha], axis=1)
    logaf = jnp.log(af)
    flow = jnp.stack([fflow, bflow], axis=1).reshape(B, 2, 2, PR, 128)
    metric2 = (jnp.concatenate([fmetric, bmetric], axis=1)
               + logaf[:, :, None, None]).reshape(B, 2, PR, 128)

    cw, idx = _prep(flow, metric2)
    tpm = _pixel_major(ftensor, btensor)
    acc = _sc_splat_call()(tpm, cw, idx)
    acc8 = acc.reshape(B, G, HW * 16 // 128, 128)
    rec = _norm(acc8)
    out = rec.reshape(B, 6, HW, 16).transpose(0, 1, 3, 2)
    return out.reshape(B, C, H, W)


# speedup vs baseline: 1.7218x; 1.7218x over previous
import functools

import jax
import jax.numpy as jnp
from jax import lax
from jax.experimental import pallas as pl
from jax.experimental.pallas import tpu as pltpu
from jax.experimental.pallas import tpu_sc as plsc

B, C, H, W = 2, 96, 256, 256
HW = H * W
G = 7
NTILES = 16
RPT = HW // NTILES
CHUNK = 128
SCHUNK = 512
NSUB = SCHUNK // CHUNK
HB2 = 8
HB3 = 32
UNROLL = 16
PR = HW // 128


def _prep_body(flow_ref, metric_ref, cw_ref, idx_ref):
    ii = lax.broadcasted_iota(jnp.int32, (PR, 128), 0)
    jj = lax.broadcasted_iota(jnp.int32, (PR, 128), 1)
    gy = (ii // 2).astype(jnp.float32)
    gx = ((ii % 2) * 128 + jj).astype(jnp.float32)
    fx = gx + flow_ref[0, 0, 0]
    fy = gy + flow_ref[0, 0, 1]
    x0 = jnp.floor(fx)
    y0 = jnp.floor(fy)
    x0i = x0.astype(jnp.int32)
    y0i = y0.astype(jnp.int32)
    wexp = jnp.exp(metric_ref[0, 0])
    for k in range(4):
        dx, dy = k % 2, k // 2
        xi = x0i + dx
        yi = y0i + dy
        wx = (x0 + 1.0 - fx) if dx == 0 else (fx - x0)
        wy = (y0 + 1.0 - fy) if dy == 0 else (fy - y0)
        valid = (xi >= 0) & (xi < W) & (yi >= 0) & (yi < H)
        cw_ref[0, 0, k] = jnp.where(valid, wx * wy * wexp, 0.0)
        idx_ref[0, 0, k] = jnp.clip(yi, 0, H - 1) * W + jnp.clip(xi, 0, W - 1)


def _prep(flow, metric2):
    return pl.pallas_call(
        _prep_body,
        grid=(B, 2),
        in_specs=[
            pl.BlockSpec((1, 1, 2, PR, 128), lambda b, d: (b, d, 0, 0, 0)),
            pl.BlockSpec((1, 1, PR, 128), lambda b, d: (b, d, 0, 0)),
        ],
        out_specs=[
            pl.BlockSpec((1, 1, 4, PR, 128), lambda b, d: (b, d, 0, 0, 0)),
            pl.BlockSpec((1, 1, 4, PR, 128), lambda b, d: (b, d, 0, 0, 0)),
        ],
        out_shape=[
            jax.ShapeDtypeStruct((B, 2, 4, PR, 128), jnp.float32),
            jax.ShapeDtypeStruct((B, 2, 4, PR, 128), jnp.int32),
        ],
    )(flow, metric2)


def _pm_body(f_ref, b_ref, out_ref):
    pad = jnp.where(
        lax.broadcasted_iota(jnp.int32, (32, HB2, W), 0) == 0, 1.0, 0.0)
    for d, src in enumerate((f_ref, b_ref)):
        full = jnp.concatenate([src[0], pad], axis=0)
        for h in range(HB2):
            out_ref[0, d, pl.ds(h * W, W), :] = full[:, h, :].T


def _pixel_major(ftensor, btensor):
    return pl.pallas_call(
        _pm_body,
        grid=(B, H // HB2),
        in_specs=[
            pl.BlockSpec((1, C, HB2, W), lambda b, hb: (b, 0, hb, 0)),
            pl.BlockSpec((1, C, HB2, W), lambda b, hb: (b, 0, hb, 0)),
        ],
        out_specs=pl.BlockSpec((1, 2, HB2 * W, 128),
                               lambda b, hb: (b, 0, hb, 0)),
        out_shape=jax.ShapeDtypeStruct((B, 2, HW, 128), jnp.float32),
    )(ftensor, btensor)


def _sc_splat_body(tpm_hbm, cw_hbm, idx_hbm, out_hbm,
                   rows_v, sb0, sb1, w_v, idx_v, zero_v,
                   acc_sh, sem0, sem1):
    b = lax.axis_index("c")
    s = lax.axis_index("s")
    base = s * RPT
    sbufs = (sb0, sb1)
    sems = (sem0, sem1)

    def zfill(i, _):
        zero_v[i, :] = jnp.zeros((16,), jnp.float32)
        return 0
    lax.fori_loop(0, SCHUNK, zfill, 0)

    def per_g(g, _):
        def zloop(z, _):
            pltpu.sync_copy(zero_v, acc_sh.at[pl.ds(base + z * SCHUNK, SCHUNK)])
            return 0
        lax.fori_loop(0, RPT // SCHUNK, zloop, 0)
        plsc.subcore_barrier()

        def per_dc(dc, _):
            d = dc // (RPT // SCHUNK)
            c2 = dc % (RPT // SCHUNK)
            off = base + c2 * SCHUNK
            pltpu.sync_copy(
                tpm_hbm.at[b, d, pl.ds(off, SCHUNK), pl.ds(g * 16, 16)],
                rows_v)
            pltpu.sync_copy(
                cw_hbm.at[b, d, :, pl.ds(off // 128, NSUB)], w_v)
            pltpu.sync_copy(
                idx_hbm.at[b, d, :, pl.ds(off // 128, NSUB)], idx_v)

            pend = [None, None]
            for k in range(4):
                sb = sbufs[k % 2]
                if pend[k % 2] is not None:
                    for dsc in pend[k % 2]:
                        dsc.wait()

                @plsc.parallel_loop(0, SCHUNK // UNROLL, 1, unroll=2)
                def scale16(i, sb=sb, k=k):
                    r = i * UNROLL
                    wv = w_v[k, r // 128, pl.ds(r % 128, UNROLL)]
                    vals = [rows_v[r + u, :] for u in range(UNROLL)]
                    prods = [vals[u] * wv[u] for u in range(UNROLL)]
                    for u in range(UNROLL):
                        sb[r + u, :] = prods[u]

                pend[k % 2] = [
                    pltpu.async_copy(sb.at[pl.ds(j * CHUNK, CHUNK)],
                                     acc_sh.at[idx_v.at[k, j]],
                                     sems[k % 2], add=True)
                    for j in range(NSUB)
                ]

            for q in range(2):
                for dsc in pend[q]:
                    dsc.wait()
            return 0

        lax.fori_loop(0, 2 * (RPT // SCHUNK), per_dc, 0)
        plsc.subcore_barrier()

        def dloop(z, _):
            pltpu.sync_copy(acc_sh.at[pl.ds(base + z * SCHUNK, SCHUNK)],
                            out_hbm.at[b, g, pl.ds(base + z * SCHUNK, SCHUNK)])
            return 0
        lax.fori_loop(0, RPT // SCHUNK, dloop, 0)
        plsc.subcore_barrier()
        return 0

    lax.fori_loop(0, G, per_g, 0)


@functools.cache
def _sc_splat_call():
    mesh = plsc.VectorSubcoreMesh(core_axis_name="c", subcore_axis_name="s")
    return pl.kernel(
        _sc_splat_body,
        out_type=jax.ShapeDtypeStruct((B, G, HW, 16), jnp.float32),
        mesh=mesh,
        scratch_types=[
            pltpu.VMEM((SCHUNK, 16), jnp.float32),
            pltpu.VMEM((SCHUNK, 16), jnp.float32),
            pltpu.VMEM((SCHUNK, 16), jnp.float32),
            pltpu.VMEM((4, NSUB, 128), jnp.float32),
            pltpu.VMEM((4, NSUB, 128), jnp.int32),
            pltpu.VMEM((SCHUNK, 16), jnp.float32),
            pltpu.VMEM_SHARED((HW, 16), jnp.float32),
            pltpu.SemaphoreType.DMA,
            pltpu.SemaphoreType.DMA,
        ],
        compiler_params=pltpu.CompilerParams(use_tc_tiling_on_sc=False),
    )


PB = 512


def _norm_body(acc_ref, nacc_ref, out_ref):
    lane = lax.broadcasted_iota(jnp.int32, (PB, 128), 1) % 16
    nb = nacc_ref[0, 0]
    for sh in (1, 2, 4, 8):
        nb = jnp.where(lane >= sh, jnp.roll(nb, sh, axis=1), nb)
    nb = jnp.where(nb == 0.0, 1.0, nb)
    out_ref[0, 0] = acc_ref[0, 0] / nb


def _norm(acc8):
    return pl.pallas_call(
        _norm_body,
        grid=(B, 6, (HW * 16 // 128) // PB),
        in_specs=[
            pl.BlockSpec((1, 1, PB, 128), lambda b, g, p: (b, g, p, 0)),
            pl.BlockSpec((1, 1, PB, 128), lambda b, g, p: (b, 6, p, 0)),
        ],
        out_specs=pl.BlockSpec((1, 1, PB, 128), lambda b, g, p: (b, g, p, 0)),
        out_shape=jax.ShapeDtypeStruct((B, 6, HW * 16 // 128, 128),
                                       jnp.float32),
    )(acc8, acc8)


def kernel(ftensor, fflow, fmetric, btensor, bflow, bmetric, t, N):
    alpha = (t / N).astype(jnp.float32)
    af = jnp.stack([1.0 - alpha, alpha], axis=1)
    logaf = jnp.log(af)
    flow = jnp.stack([fflow, bflow], axis=1).reshape(B, 2, 2, PR, 128)
    metric2 = (jnp.concatenate([fmetric, bmetric], axis=1)
               + logaf[:, :, None, None]).reshape(B, 2, PR, 128)

    cw, idx = _prep(flow, metric2)
    tpm = _pixel_major(ftensor, btensor)
    acc = _sc_splat_call()(tpm, cw, idx)
    acc8 = acc.reshape(B, G, HW * 16 // 128, 128)
    rec = _norm(acc8)
    out = rec.reshape(B, 6, HW, 16).transpose(0, 1, 3, 2)
    return out.reshape(B, C, H, W)


# speedup vs baseline: 1.8926x; 1.0992x over previous
import functools

import jax
import jax.numpy as jnp
from jax import lax
from jax.experimental import pallas as pl
from jax.experimental.pallas import tpu as pltpu
from jax.experimental.pallas import tpu_sc as plsc

B, C, H, W = 2, 96, 256, 256
HW = H * W
G = 7
NTILES = 16
RPT = HW // NTILES
CHUNK = 128
SCHUNK = 512
NSUB = SCHUNK // CHUNK
HB2 = 8
HB3 = 32
UNROLL = 16
PR = HW // 128


def _prep_body(flow_ref, metric_ref, cw_ref, idx_ref):
    ii = lax.broadcasted_iota(jnp.int32, (PR, 128), 0)
    jj = lax.broadcasted_iota(jnp.int32, (PR, 128), 1)
    gy = (ii // 2).astype(jnp.float32)
    gx = ((ii % 2) * 128 + jj).astype(jnp.float32)
    fx = gx + flow_ref[0, 0, 0]
    fy = gy + flow_ref[0, 0, 1]
    x0 = jnp.floor(fx)
    y0 = jnp.floor(fy)
    x0i = x0.astype(jnp.int32)
    y0i = y0.astype(jnp.int32)
    wexp = jnp.exp(metric_ref[0, 0])
    for k in range(4):
        dx, dy = k % 2, k // 2
        xi = x0i + dx
        yi = y0i + dy
        wx = (x0 + 1.0 - fx) if dx == 0 else (fx - x0)
        wy = (y0 + 1.0 - fy) if dy == 0 else (fy - y0)
        valid = (xi >= 0) & (xi < W) & (yi >= 0) & (yi < H)
        cw_ref[0, 0, k] = jnp.where(valid, wx * wy * wexp, 0.0)
        idx_ref[0, 0, k] = jnp.clip(yi, 0, H - 1) * W + jnp.clip(xi, 0, W - 1)


def _prep(flow, metric2):
    return pl.pallas_call(
        _prep_body,
        grid=(B, 2),
        in_specs=[
            pl.BlockSpec((1, 1, 2, PR, 128), lambda b, d: (b, d, 0, 0, 0)),
            pl.BlockSpec((1, 1, PR, 128), lambda b, d: (b, d, 0, 0)),
        ],
        out_specs=[
            pl.BlockSpec((1, 1, 4, PR, 128), lambda b, d: (b, d, 0, 0, 0)),
            pl.BlockSpec((1, 1, 4, PR, 128), lambda b, d: (b, d, 0, 0, 0)),
        ],
        out_shape=[
            jax.ShapeDtypeStruct((B, 2, 4, PR, 128), jnp.float32),
            jax.ShapeDtypeStruct((B, 2, 4, PR, 128), jnp.int32),
        ],
    )(flow, metric2)


def _pm_body(f_ref, b_ref, out_ref):
    pad = jnp.where(
        lax.broadcasted_iota(jnp.int32, (32, HB2, W), 0) == 0, 1.0, 0.0)
    for d, src in enumerate((f_ref, b_ref)):
        full = jnp.concatenate([src[0], pad], axis=0)
        for h in range(HB2):
            out_ref[0, d, pl.ds(h * W, W), :] = full[:, h, :].T


def _pixel_major(ftensor, btensor):
    return pl.pallas_call(
        _pm_body,
        grid=(B, H // HB2),
        in_specs=[
            pl.BlockSpec((1, C, HB2, W), lambda b, hb: (b, 0, hb, 0)),
            pl.BlockSpec((1, C, HB2, W), lambda b, hb: (b, 0, hb, 0)),
        ],
        out_specs=pl.BlockSpec((1, 2, HB2 * W, 128),
                               lambda b, hb: (b, 0, hb, 0)),
        out_shape=jax.ShapeDtypeStruct((B, 2, HW, 128), jnp.float32),
    )(ftensor, btensor)


def _sc_splat_body(tpm_hbm, cw_hbm, idx_hbm, out_hbm,
                   rows_v, sb0, sb1, w_v, idx_v, zero_v,
                   acc_sh, sem0, sem1):
    b = lax.axis_index("c")
    s = lax.axis_index("s")
    base = s * RPT
    sbufs = (sb0, sb1)
    sems = (sem0, sem1)

    def zfill(i, _):
        zero_v[i, :] = jnp.zeros((16,), jnp.float32)
        return 0
    lax.fori_loop(0, SCHUNK, zfill, 0)

    def per_g(g, _):
        def zloop(z, _):
            pltpu.sync_copy(zero_v, acc_sh.at[pl.ds(base + z * SCHUNK, SCHUNK)])
            return 0
        lax.fori_loop(0, RPT // SCHUNK, zloop, 0)
        plsc.subcore_barrier()

        def per_dc(dc, _):
            d = dc // (RPT // SCHUNK)
            c2 = dc % (RPT // SCHUNK)
            off = base + c2 * SCHUNK
            pltpu.sync_copy(
                tpm_hbm.at[b, d, pl.ds(off, SCHUNK), pl.ds(g * 16, 16)],
                rows_v)
            pltpu.sync_copy(
                cw_hbm.at[b, d, :, pl.ds(off // 128, NSUB)], w_v)
            pltpu.sync_copy(
                idx_hbm.at[b, d, :, pl.ds(off // 128, NSUB)], idx_v)

            pend = [None, None]
            for k in range(4):
                sb = sbufs[k % 2]
                if pend[k % 2] is not None:
                    for dsc in pend[k % 2]:
                        dsc.wait()

                @plsc.parallel_loop(0, SCHUNK // UNROLL, 1, unroll=2)
                def scale16(i, sb=sb, k=k):
                    r = i * UNROLL
                    wv = w_v[k, r // 128, pl.ds(r % 128, UNROLL)]
                    vals = [rows_v[r + u, :] for u in range(UNROLL)]
                    prods = [vals[u] * wv[u] for u in range(UNROLL)]
                    for u in range(UNROLL):
                        sb[r + u, :] = prods[u]

                pend[k % 2] = [
                    pltpu.async_copy(sb.at[pl.ds(j * CHUNK, CHUNK)],
                                     acc_sh.at[idx_v.at[k, j]],
                                     sems[k % 2], add=True)
                    for j in range(NSUB)
                ]

            for q in range(2):
                for dsc in pend[q]:
                    dsc.wait()
            return 0

        lax.fori_loop(0, 2 * (RPT // SCHUNK), per_dc, 0)
        plsc.subcore_barrier()

        def dloop(z, _):
            pltpu.sync_copy(acc_sh.at[pl.ds(base + z * SCHUNK, SCHUNK)],
                            out_hbm.at[b, g, pl.ds(base + z * SCHUNK, SCHUNK)])
            return 0
        lax.fori_loop(0, RPT // SCHUNK, dloop, 0)
        plsc.subcore_barrier()
        return 0

    lax.fori_loop(0, G, per_g, 0)


@functools.cache
def _sc_splat_call():
    mesh = plsc.VectorSubcoreMesh(core_axis_name="c", subcore_axis_name="s")
    return pl.kernel(
        _sc_splat_body,
        out_type=jax.ShapeDtypeStruct((B, G, HW, 16), jnp.float32),
        mesh=mesh,
        scratch_types=[
            pltpu.VMEM((SCHUNK, 16), jnp.float32),
            pltpu.VMEM((SCHUNK, 16), jnp.float32),
            pltpu.VMEM((SCHUNK, 16), jnp.float32),
            pltpu.VMEM((4, NSUB, 128), jnp.float32),
            pltpu.VMEM((4, NSUB, 128), jnp.int32),
            pltpu.VMEM((SCHUNK, 16), jnp.float32),
            pltpu.VMEM_SHARED((HW, 16), jnp.float32),
            pltpu.SemaphoreType.DMA,
            pltpu.SemaphoreType.DMA,
        ],
        compiler_params=pltpu.CompilerParams(use_tc_tiling_on_sc=False),
    )


PB = 512


def _norm_body(acc_ref, out_ref):
    lane = lax.broadcasted_iota(jnp.int32, (PB, 128), 1) % 16
    nb = acc_ref[0, 6]
    for sh in (1, 2, 4, 8):
        nb = jnp.where(lane >= sh, jnp.roll(nb, sh, axis=1), nb)
    rec = jnp.where(nb == 0.0, 1.0, nb)
    for g in range(6):
        out_ref[0, g] = acc_ref[0, g] / rec


def _norm(acc8):
    return pl.pallas_call(
        _norm_body,
        grid=(B, (HW * 16 // 128) // PB),
        in_specs=[
            pl.BlockSpec((1, G, PB, 128), lambda b, p: (b, 0, p, 0)),
        ],
        out_specs=pl.BlockSpec((1, 6, PB, 128), lambda b, p: (b, 0, p, 0)),
        out_shape=jax.ShapeDtypeStruct((B, 6, HW * 16 // 128, 128),
                                       jnp.float32),
    )(acc8)


def kernel(ftensor, fflow, fmetric, btensor, bflow, bmetric, t, N):
    alpha = (t / N).astype(jnp.float32)
    af = jnp.stack([1.0 - alpha, alpha], axis=1)
    logaf = jnp.log(af)
    flow = jnp.stack([fflow, bflow], axis=1).reshape(B, 2, 2, PR, 128)
    metric2 = (jnp.concatenate([fmetric, bmetric], axis=1)
               + logaf[:, :, None, None]).reshape(B, 2, PR, 128)

    cw, idx = _prep(flow, metric2)
    tpm = _pixel_major(ftensor, btensor)
    acc = _sc_splat_call()(tpm, cw, idx)
    acc8 = acc.reshape(B, G, HW * 16 // 128, 128)
    rec = _norm(acc8)
    out = rec.reshape(B, 6, HW, 16).transpose(0, 1, 3, 2)
    return out.reshape(B, C, H, W)


# speedup vs baseline: 2.3906x; 1.2631x over previous
import functools

import jax
import jax.numpy as jnp
from jax import lax
from jax.experimental import pallas as pl
from jax.experimental.pallas import tpu as pltpu
from jax.experimental.pallas import tpu_sc as plsc

B, C, H, W = 2, 96, 256, 256
HW = H * W
G = 7
NTILES = 16
RPT = HW // NTILES
CHUNK = 128
SCHUNK = 512
NSUB = SCHUNK // CHUNK
HB2 = 8
HB3 = 32
UNROLL = 16
PR = HW // 128


def _prep_body(flow_ref, metric_ref, cw_ref, idx_ref):
    ii = lax.broadcasted_iota(jnp.int32, (PR, 128), 0)
    jj = lax.broadcasted_iota(jnp.int32, (PR, 128), 1)
    gy = (ii // 2).astype(jnp.float32)
    gx = ((ii % 2) * 128 + jj).astype(jnp.float32)
    fx = gx + flow_ref[0, 0, 0]
    fy = gy + flow_ref[0, 0, 1]
    x0 = jnp.floor(fx)
    y0 = jnp.floor(fy)
    x0i = x0.astype(jnp.int32)
    y0i = y0.astype(jnp.int32)
    wexp = jnp.exp(metric_ref[0, 0])
    for k in range(4):
        dx, dy = k % 2, k // 2
        xi = x0i + dx
        yi = y0i + dy
        wx = (x0 + 1.0 - fx) if dx == 0 else (fx - x0)
        wy = (y0 + 1.0 - fy) if dy == 0 else (fy - y0)
        valid = (xi >= 0) & (xi < W) & (yi >= 0) & (yi < H)
        cw_ref[0, 0, k] = jnp.where(valid, wx * wy * wexp, 0.0)
        idx_ref[0, 0, k] = jnp.clip(yi, 0, H - 1) * W + jnp.clip(xi, 0, W - 1)


def _prep(flow, metric2):
    return pl.pallas_call(
        _prep_body,
        grid=(B, 2),
        in_specs=[
            pl.BlockSpec((1, 1, 2, PR, 128), lambda b, d: (b, d, 0, 0, 0)),
            pl.BlockSpec((1, 1, PR, 128), lambda b, d: (b, d, 0, 0)),
        ],
        out_specs=[
            pl.BlockSpec((1, 1, 4, PR, 128), lambda b, d: (b, d, 0, 0, 0)),
            pl.BlockSpec((1, 1, 4, PR, 128), lambda b, d: (b, d, 0, 0, 0)),
        ],
        out_shape=[
            jax.ShapeDtypeStruct((B, 2, 4, PR, 128), jnp.float32),
            jax.ShapeDtypeStruct((B, 2, 4, PR, 128), jnp.int32),
        ],
    )(flow, metric2)


def _pm_body(f_ref, b_ref, out_ref):
    pad = jnp.where(
        lax.broadcasted_iota(jnp.int32, (32, HB2, W), 0) == 0, 1.0, 0.0)
    for d, src in enumerate((f_ref, b_ref)):
        full = jnp.concatenate([src[0], pad], axis=0)
        for h in range(HB2):
            out_ref[0, d, pl.ds(h * W, W), :] = full[:, h, :].T


def _pixel_major(ftensor, btensor):
    return pl.pallas_call(
        _pm_body,
        grid=(B, H // HB2),
        in_specs=[
            pl.BlockSpec((1, C, HB2, W), lambda b, hb: (b, 0, hb, 0)),
            pl.BlockSpec((1, C, HB2, W), lambda b, hb: (b, 0, hb, 0)),
        ],
        out_specs=pl.BlockSpec((1, 2, HB2 * W, 128),
                               lambda b, hb: (b, 0, hb, 0)),
        out_shape=jax.ShapeDtypeStruct((B, 2, HW, 128), jnp.float32),
    )(ftensor, btensor)


NDC = 2 * (RPT // SCHUNK)


def _sc_splat_body(tpm_hbm, cw_hbm, idx_hbm, out_hbm,
                   rowsA, rowsB, sb0, sb1, wA, wB, idxA, idxB, zero_v,
                   acc_sh, sem0, sem1, semA, semB):
    b = lax.axis_index("c")
    s = lax.axis_index("s")
    base = s * RPT
    sbufs = (sb0, sb1)
    sems = (sem0, sem1)

    def zfill(i, _):
        zero_v[i, :] = jnp.zeros((16,), jnp.float32)
        return 0
    lax.fori_loop(0, SCHUNK, zfill, 0)

    def per_g(g, _):
        def zloop(z, _):
            pltpu.sync_copy(zero_v, acc_sh.at[pl.ds(base + z * SCHUNK, SCHUNK)])
            return 0
        lax.fori_loop(0, RPT // SCHUNK, zloop, 0)
        plsc.subcore_barrier()

        def src_slices(dc):
            d = dc // (RPT // SCHUNK)
            c2 = dc % (RPT // SCHUNK)
            off = base + c2 * SCHUNK
            return (tpm_hbm.at[b, d, pl.ds(off, SCHUNK), pl.ds(g * 16, 16)],
                    cw_hbm.at[b, d, :, pl.ds(off // 128, NSUB)],
                    idx_hbm.at[b, d, :, pl.ds(off // 128, NSUB)])

        def stage(dc, rows_b, w_b, idx_b, sem):
            t, c, i = src_slices(dc)
            pltpu.async_copy(t, rows_b, sem)
            pltpu.async_copy(c, w_b, sem)
            pltpu.async_copy(i, idx_b, sem)

        def wait_stage(rows_b, w_b, idx_b, sem):
            t, c, i = src_slices(0)
            pltpu.make_async_copy(t, rows_b, sem).wait()
            pltpu.make_async_copy(c, w_b, sem).wait()
            pltpu.make_async_copy(i, idx_b, sem).wait()

        def process(rows_b, w_b, idx_b):
            pend = [None, None]
            for k in range(4):
                sb = sbufs[k % 2]
                if pend[k % 2] is not None:
                    for dsc in pend[k % 2]:
                        dsc.wait()

                @plsc.parallel_loop(0, SCHUNK // UNROLL, 1, unroll=2)
                def scale16(i, sb=sb, k=k):
                    r = i * UNROLL
                    wv = w_b[k, r // 128, pl.ds(r % 128, UNROLL)]
                    vals = [rows_b[r + u, :] for u in range(UNROLL)]
                    prods = [vals[u] * wv[u] for u in range(UNROLL)]
                    for u in range(UNROLL):
                        sb[r + u, :] = prods[u]

                pend[k % 2] = [
                    pltpu.async_copy(sb.at[pl.ds(j * CHUNK, CHUNK)],
                                     acc_sh.at[idx_b.at[k, j]],
                                     sems[k % 2], add=True)
                    for j in range(NSUB)
                ]

            for q in range(2):
                for dsc in pend[q]:
                    dsc.wait()

        stage(0, rowsA, wA, idxA, semA)

        def per_pair(p, _):
            dc0 = 2 * p
            stage(dc0 + 1, rowsB, wB, idxB, semB)
            wait_stage(rowsA, wA, idxA, semA)
            process(rowsA, wA, idxA)

            @pl.when(dc0 + 2 < NDC)
            def _():
                stage(dc0 + 2, rowsA, wA, idxA, semA)

            wait_stage(rowsB, wB, idxB, semB)
            process(rowsB, wB, idxB)
            return 0

        lax.fori_loop(0, NDC // 2, per_pair, 0)
        plsc.subcore_barrier()

        def dloop(z, _):
            pltpu.sync_copy(acc_sh.at[pl.ds(base + z * SCHUNK, SCHUNK)],
                            out_hbm.at[b, g, pl.ds(base + z * SCHUNK, SCHUNK)])
            return 0
        lax.fori_loop(0, RPT // SCHUNK, dloop, 0)
        plsc.subcore_barrier()
        return 0

    lax.fori_loop(0, G, per_g, 0)


@functools.cache
def _sc_splat_call():
    mesh = plsc.VectorSubcoreMesh(core_axis_name="c", subcore_axis_name="s")
    return pl.kernel(
        _sc_splat_body,
        out_type=jax.ShapeDtypeStruct((B, G, HW, 16), jnp.float32),
        mesh=mesh,
        scratch_types=[
            pltpu.VMEM((SCHUNK, 16), jnp.float32),
            pltpu.VMEM((SCHUNK, 16), jnp.float32),
            pltpu.VMEM((SCHUNK, 16), jnp.float32),
            pltpu.VMEM((SCHUNK, 16), jnp.float32),
            pltpu.VMEM((4, NSUB, 128), jnp.float32),
            pltpu.VMEM((4, NSUB, 128), jnp.float32),
            pltpu.VMEM((4, NSUB, 128), jnp.int32),
            pltpu.VMEM((4, NSUB, 128), jnp.int32),
            pltpu.VMEM((SCHUNK, 16), jnp.float32),
            pltpu.VMEM_SHARED((HW, 16), jnp.float32),
            pltpu.SemaphoreType.DMA,
            pltpu.SemaphoreType.DMA,
            pltpu.SemaphoreType.DMA,
            pltpu.SemaphoreType.DMA,
        ],
        compiler_params=pltpu.CompilerParams(use_tc_tiling_on_sc=False),
    )


PB = 512


def _norm_body(acc_ref, out_ref):
    lane = lax.broadcasted_iota(jnp.int32, (PB, 128), 1) % 16
    nb = acc_ref[0, 6]
    for sh in (1, 2, 4, 8):
        nb = jnp.where(lane >= sh, jnp.roll(nb, sh, axis=1), nb)
    rec = jnp.where(nb == 0.0, 1.0, nb)
    for g in range(6):
        out_ref[0, g] = acc_ref[0, g] / rec


def _norm(acc8):
    return pl.pallas_call(
        _norm_body,
        grid=(B, (HW * 16 // 128) // PB),
        in_specs=[
            pl.BlockSpec((1, G, PB, 128), lambda b, p: (b, 0, p, 0)),
        ],
        out_specs=pl.BlockSpec((1, 6, PB, 128), lambda b, p: (b, 0, p, 0)),
        out_shape=jax.ShapeDtypeStruct((B, 6, HW * 16 // 128, 128),
                                       jnp.float32),
    )(acc8)


def kernel(ftensor, fflow, fmetric, btensor, bflow, bmetric, t, N):
    alpha = (t / N).astype(jnp.float32)
    af = jnp.stack([1.0 - alpha, alpha], axis=1)
    logaf = jnp.log(af)
    flow = jnp.stack([fflow, bflow], axis=1).reshape(B, 2, 2, PR, 128)
    metric2 = (jnp.concatenate([fmetric, bmetric], axis=1)
               + logaf[:, :, None, None]).reshape(B, 2, PR, 128)

    cw, idx = _prep(flow, metric2)
    tpm = _pixel_major(ftensor, btensor)
    acc = _sc_splat_call()(tpm, cw, idx)
    acc8 = acc.reshape(B, G, HW * 16 // 128, 128)
    rec = _norm(acc8)
    out = rec.reshape(B, 6, HW, 16).transpose(0, 1, 3, 2)
    return out.reshape(B, C, H, W)
